# Initial kernel scaffold; baseline (speedup 1.0000x reference)
import jax
import jax.numpy as jnp
from jax.experimental import pallas as pl


def kernel(input, target):
    raise NotImplementedError("write your pallas kernel here")


# speedup vs baseline: 1.8634x; 1.8634x over previous
import jax
import jax.numpy as jnp
from jax.experimental import pallas as pl
from jax.experimental.pallas import tpu as pltpu

ZERO_WEIGHT = 2.0

_ROWS = 2 * 8192
_COLS = 2048
_BLOCK_ROWS = 1024
_GRID = _ROWS // _BLOCK_ROWS


def _body(x_ref, t_ref, out_ref, acc_ref):
    i = pl.program_id(0)

    @pl.when(i == 0)
    def _init():
        acc_ref[0] = 0.0
        acc_ref[1] = 0.0
        acc_ref[2] = 0.0

    x = x_ref[...]
    t = t_ref[...]
    d = x - t
    sq = d * d
    zero = t == 0.0
    acc_ref[0] += jnp.sum(sq)
    acc_ref[1] += jnp.sum(jnp.where(zero, sq, 0.0))
    acc_ref[2] += jnp.sum(zero.astype(jnp.float32))

    @pl.when(i == pl.num_programs(0) - 1)
    def _fini():
        n_total = float(_ROWS * _COLS)
        s_all = acc_ref[0]
        s_z = acc_ref[1]
        n_z = acc_ref[2]
        n_uz = n_total - n_z
        z_ratio = n_z / n_total
        loss_comp = s_all / n_total
        loss_z = s_z / jnp.maximum(n_z, 1.0)
        loss_uz = (s_all - s_z) / jnp.maximum(n_uz, 1.0)
        loss = loss_z * z_ratio * ZERO_WEIGHT + loss_uz * (1.0 - z_ratio)
        out_ref[0] = loss * (loss_comp / loss)


def kernel(input, target):
    x = input.reshape(_ROWS, _COLS)
    t = target.reshape(_ROWS, _COLS)
    out = pl.pallas_call(
        _body,
        grid=(_GRID,),
        in_specs=[
            pl.BlockSpec((_BLOCK_ROWS, _COLS), lambda i: (i, 0)),
            pl.BlockSpec((_BLOCK_ROWS, _COLS), lambda i: (i, 0)),
        ],
        out_specs=pl.BlockSpec(memory_space=pltpu.SMEM),
        out_shape=jax.ShapeDtypeStruct((1,), jnp.float32),
        scratch_shapes=[pltpu.SMEM((3,), jnp.float32)],
    )(x, t)
    return out[0]
